# Initial kernel scaffold; baseline (speedup 1.0000x reference)
import jax
import jax.numpy as jnp
from jax.experimental import pallas as pl


def kernel(features, nodes, to_neighs):
    raise NotImplementedError("write your pallas kernel here")


# speedup vs baseline: 3.5597x; 3.5597x over previous
import functools

import jax
import jax.numpy as jnp
from jax import lax
from jax.experimental import pallas as pl
from jax.experimental.pallas import tpu as pltpu
from jax.experimental.pallas import tpu_sc as plsc

D = 128
S = 10
L = 16
NW = 32
C = 32
R = C * S
CHUNKS = 49
PER_TILE = C * CHUNKS
BPAD = PER_TILE * NW


def _sc_mean(features, idx_flat):
    mesh = plsc.VectorSubcoreMesh(core_axis_name="c", subcore_axis_name="s")

    @functools.partial(
        pl.kernel,
        mesh=mesh,
        out_type=jax.ShapeDtypeStruct((BPAD, D), jnp.float32),
        scratch_types=[
            pltpu.VMEM((R,), jnp.int32),
            pltpu.VMEM((R, D), jnp.float32),
            pltpu.VMEM((C, D), jnp.float32),
            pltpu.SemaphoreType.DMA,
        ],
    )
    def k(feat_hbm, idx_hbm, out_hbm, idx_v, rows_v, out_v, sem):
        wid = lax.axis_index("s") * 2 + lax.axis_index("c")
        tile_node0 = wid * PER_TILE

        def chunk_body(ci, carry):
            node0 = tile_node0 + ci * C
            row0 = node0 * S
            pltpu.sync_copy(idx_hbm.at[pl.ds(row0, R)], idx_v)
            cps = []
            for g0, gn in ((0, 128), (128, 128), (256, 64)):
                cps.append(pltpu.async_copy(
                    feat_hbm.at[idx_v.at[pl.ds(g0, gn)]],
                    rows_v.at[pl.ds(g0, gn)],
                    sem,
                ))
            for cp in cps:
                cp.wait()

            def node_body(n, carry2):
                base = n * S
                for c in range(D // L):
                    acc = rows_v[base, pl.ds(c * L, L)]
                    for s_ in range(1, S):
                        acc = acc + rows_v[base + s_, pl.ds(c * L, L)]
                    out_v[n, pl.ds(c * L, L)] = acc * jnp.float32(0.1)
                return carry2

            lax.fori_loop(0, C, node_body, 0)
            pltpu.sync_copy(out_v, out_hbm.at[pl.ds(node0, C)])
            return carry

        lax.fori_loop(0, CHUNKS, chunk_body, 0)

    return k(features, idx_flat)


def kernel(features, nodes, to_neighs):
    b = to_neighs.shape[0]
    idx = to_neighs.astype(jnp.int32).reshape(-1)
    idx = jnp.pad(idx, (0, BPAD * S - idx.shape[0]))
    out = _sc_mean(features, idx)
    return out[:b]
